# baseline (speedup 1.0000x reference)
import jax
import jax.numpy as jnp
from jax import lax
from jax.experimental import pallas as pl
from jax.experimental.pallas import tpu as pltpu
from jax.experimental.pallas import tpu_sc as plsc

BATCH = 8192
WIDTH = 2048
LANES = 16
NUM_CORES = 2
NUM_SUBCORES = 16
NUM_TILES = NUM_CORES * NUM_SUBCORES
ROWS_PER_TILE = BATCH // NUM_TILES
CHUNK_ROWS = 16
NUM_CHUNKS = ROWS_PER_TILE // CHUNK_ROWS
CHUNK_ELEMS = CHUNK_ROWS * WIDTH


def _sc_body(x_hbm, perm_hbm, y_hbm, perm_v, in_v, out_v):
    wid = lax.axis_index("s") * NUM_CORES + lax.axis_index("c")
    base = wid * ROWS_PER_TILE * WIDTH
    pltpu.sync_copy(perm_hbm, perm_v)

    @pl.loop(0, NUM_CHUNKS)
    def _chunk_loop(chunk):
        e0 = base + chunk * CHUNK_ELEMS
        pltpu.sync_copy(x_hbm.at[pl.ds(e0, CHUNK_ELEMS)], in_v)

        @pl.loop(0, CHUNK_ROWS)
        def _row_loop(r):
            rbase = jnp.full((LANES,), r * WIDTH, jnp.int32)
            for jc in range(WIDTH // LANES):
                idx = perm_v[pl.ds(jc * LANES, LANES)] + rbase
                vals = plsc.load_gather(in_v, [idx])
                out_v[pl.ds(r * WIDTH + jc * LANES, LANES)] = vals

        pltpu.sync_copy(out_v, y_hbm.at[pl.ds(e0, CHUNK_ELEMS)])


def kernel(x, perm):
    mesh = plsc.VectorSubcoreMesh(
        core_axis_name="c", subcore_axis_name="s",
        num_cores=NUM_CORES, num_subcores=NUM_SUBCORES)
    run = pl.kernel(
        _sc_body,
        out_type=jax.ShapeDtypeStruct((BATCH * WIDTH,), jnp.float32),
        mesh=mesh,
        scratch_types=[
            pltpu.VMEM((WIDTH,), jnp.int32),
            pltpu.VMEM((CHUNK_ELEMS,), jnp.float32),
            pltpu.VMEM((CHUNK_ELEMS,), jnp.float32),
        ],
        compiler_params=pltpu.CompilerParams(needs_layout_passes=False),
    )
    y = run(x.reshape(-1), perm.astype(jnp.int32))
    return (y.reshape(BATCH, WIDTH), 0.0)


# speedup vs baseline: 1.7103x; 1.7103x over previous
import jax
import jax.numpy as jnp
from jax import lax
from jax.experimental import pallas as pl
from jax.experimental.pallas import tpu as pltpu
from jax.experimental.pallas import tpu_sc as plsc

BATCH = 8192
WIDTH = 2048
LANES = 16
NUM_CORES = 2
NUM_SUBCORES = 16
NUM_TILES = NUM_CORES * NUM_SUBCORES
ROWS_PER_TILE = BATCH // NUM_TILES
CHUNK_ROWS = 8
NUM_CHUNKS = ROWS_PER_TILE // CHUNK_ROWS
CHUNK_ELEMS = CHUNK_ROWS * WIDTH


def _sc_body(x_hbm, perm_hbm, y_hbm, perm_v, in0, in1, out0, out1,
             sin0, sin1, sout0, sout1):
    wid = lax.axis_index("s") * NUM_CORES + lax.axis_index("c")
    base = wid * ROWS_PER_TILE * WIDTH
    pltpu.sync_copy(perm_hbm, perm_v)

    def in_slice(c):
        return x_hbm.at[pl.ds(base + c * CHUNK_ELEMS, CHUNK_ELEMS)]

    def out_slice(c):
        return y_hbm.at[pl.ds(base + c * CHUNK_ELEMS, CHUNK_ELEMS)]

    def permute_chunk(in_ref, out_ref):
        @pl.loop(0, WIDTH // LANES)
        def _col_loop(jc):
            pv = perm_v[pl.ds(jc * LANES, LANES)]
            off = jc * LANES
            for r in range(CHUNK_ROWS):
                vals = plsc.load_gather(in_ref, [pv + jnp.int32(r * WIDTH)])
                out_ref[pl.ds(off + r * WIDTH, LANES)] = vals

    pltpu.async_copy(in_slice(0), in0, sin0)

    @pl.loop(0, NUM_CHUNKS, step=2)
    def _chunk_loop(c):
        pltpu.async_copy(in_slice(c + 1), in1, sin1)
        pltpu.make_async_copy(in_slice(c), in0, sin0).wait()

        @pl.when(c >= 2)
        def _drain_out0():
            pltpu.make_async_copy(out0, out_slice(c), sout0).wait()

        permute_chunk(in0, out0)
        pltpu.async_copy(out0, out_slice(c), sout0)

        @pl.when(c + 2 < NUM_CHUNKS)
        def _prefetch_in0():
            pltpu.async_copy(in_slice(c + 2), in0, sin0)

        pltpu.make_async_copy(in_slice(c + 1), in1, sin1).wait()

        @pl.when(c >= 2)
        def _drain_out1():
            pltpu.make_async_copy(out1, out_slice(c + 1), sout1).wait()

        permute_chunk(in1, out1)
        pltpu.async_copy(out1, out_slice(c + 1), sout1)

    pltpu.make_async_copy(out0, out_slice(NUM_CHUNKS - 2), sout0).wait()
    pltpu.make_async_copy(out1, out_slice(NUM_CHUNKS - 1), sout1).wait()


def kernel(x, perm):
    mesh = plsc.VectorSubcoreMesh(
        core_axis_name="c", subcore_axis_name="s",
        num_cores=NUM_CORES, num_subcores=NUM_SUBCORES)
    run = pl.kernel(
        _sc_body,
        out_type=jax.ShapeDtypeStruct((BATCH * WIDTH,), jnp.float32),
        mesh=mesh,
        scratch_types=[
            pltpu.VMEM((WIDTH,), jnp.int32),
            pltpu.VMEM((CHUNK_ELEMS,), jnp.float32),
            pltpu.VMEM((CHUNK_ELEMS,), jnp.float32),
            pltpu.VMEM((CHUNK_ELEMS,), jnp.float32),
            pltpu.VMEM((CHUNK_ELEMS,), jnp.float32),
            pltpu.SemaphoreType.DMA,
            pltpu.SemaphoreType.DMA,
            pltpu.SemaphoreType.DMA,
            pltpu.SemaphoreType.DMA,
        ],
        compiler_params=pltpu.CompilerParams(needs_layout_passes=False),
    )
    y = run(x.reshape(-1), perm.astype(jnp.int32))
    return (y.reshape(BATCH, WIDTH), 0.0)


# speedup vs baseline: 2.7698x; 1.6195x over previous
import jax
import jax.numpy as jnp
from jax import lax
from jax.experimental import pallas as pl
from jax.experimental.pallas import tpu as pltpu
from jax.experimental.pallas import tpu_sc as plsc

BATCH = 8192
WIDTH = 2048
LANES = 16
NUM_CORES = 2
NUM_SUBCORES = 16
NUM_TILES = NUM_CORES * NUM_SUBCORES
ROWS_PER_TILE = BATCH // NUM_TILES
CHUNK_ROWS = 8
NUM_CHUNKS = ROWS_PER_TILE // CHUNK_ROWS
CHUNK_ELEMS = CHUNK_ROWS * WIDTH


def _sc_body(x_hbm, perm_hbm, y_hbm, perm_v, in0, in1, out0, out1,
             sin0, sin1, sout0, sout1):
    wid = lax.axis_index("s") * NUM_CORES + lax.axis_index("c")
    base = wid * ROWS_PER_TILE * WIDTH
    pltpu.sync_copy(perm_hbm, perm_v)

    def in_slice(c):
        return x_hbm.at[pl.ds(base + c * CHUNK_ELEMS, CHUNK_ELEMS)]

    def out_slice(c):
        return y_hbm.at[pl.ds(base + c * CHUNK_ELEMS, CHUNK_ELEMS)]

    def permute_chunk(in_ref, out_ref):
        @plsc.parallel_loop(0, WIDTH // LANES, unroll=4)
        def _col_loop(jc):
            pv = perm_v[pl.ds(jc * LANES, LANES)]
            off = jc * LANES
            for r in range(CHUNK_ROWS):
                vals = plsc.load_gather(in_ref, [pv + jnp.int32(r * WIDTH)])
                out_ref[pl.ds(off + r * WIDTH, LANES)] = vals

    pltpu.async_copy(in_slice(0), in0, sin0)

    @pl.loop(0, NUM_CHUNKS, step=2)
    def _chunk_loop(c):
        pltpu.async_copy(in_slice(c + 1), in1, sin1)
        pltpu.make_async_copy(in_slice(c), in0, sin0).wait()

        @pl.when(c >= 2)
        def _drain_out0():
            pltpu.make_async_copy(out0, out_slice(c), sout0).wait()

        permute_chunk(in0, out0)
        pltpu.async_copy(out0, out_slice(c), sout0)

        @pl.when(c + 2 < NUM_CHUNKS)
        def _prefetch_in0():
            pltpu.async_copy(in_slice(c + 2), in0, sin0)

        pltpu.make_async_copy(in_slice(c + 1), in1, sin1).wait()

        @pl.when(c >= 2)
        def _drain_out1():
            pltpu.make_async_copy(out1, out_slice(c + 1), sout1).wait()

        permute_chunk(in1, out1)
        pltpu.async_copy(out1, out_slice(c + 1), sout1)

    pltpu.make_async_copy(out0, out_slice(NUM_CHUNKS - 2), sout0).wait()
    pltpu.make_async_copy(out1, out_slice(NUM_CHUNKS - 1), sout1).wait()


def kernel(x, perm):
    mesh = plsc.VectorSubcoreMesh(
        core_axis_name="c", subcore_axis_name="s",
        num_cores=NUM_CORES, num_subcores=NUM_SUBCORES)
    run = pl.kernel(
        _sc_body,
        out_type=jax.ShapeDtypeStruct((BATCH * WIDTH,), jnp.float32),
        mesh=mesh,
        scratch_types=[
            pltpu.VMEM((WIDTH,), jnp.int32),
            pltpu.VMEM((CHUNK_ELEMS,), jnp.float32),
            pltpu.VMEM((CHUNK_ELEMS,), jnp.float32),
            pltpu.VMEM((CHUNK_ELEMS,), jnp.float32),
            pltpu.VMEM((CHUNK_ELEMS,), jnp.float32),
            pltpu.SemaphoreType.DMA,
            pltpu.SemaphoreType.DMA,
            pltpu.SemaphoreType.DMA,
            pltpu.SemaphoreType.DMA,
        ],
        compiler_params=pltpu.CompilerParams(needs_layout_passes=False),
    )
    y = run(x.reshape(-1), perm.astype(jnp.int32))
    return (y.reshape(BATCH, WIDTH), 0.0)


# speedup vs baseline: 7.2606x; 2.6213x over previous
import jax
import jax.numpy as jnp
from jax import lax
from jax.experimental import pallas as pl
from jax.experimental.pallas import tpu as pltpu
from jax.experimental.pallas import tpu_sc as plsc

BATCH = 8192
WIDTH = 2048
LANES = 16
NUM_CORES = 2
NUM_SUBCORES = 16
NUM_TILES = NUM_CORES * NUM_SUBCORES
ROWS_PER_TILE = BATCH // NUM_TILES
CHUNK_ROWS = 8
NUM_CHUNKS = ROWS_PER_TILE // CHUNK_ROWS


def _sc_body(x_hbm, perm_hbm, y_hbm, perm_v, in0, in1, out0, out1,
             sin0, sin1, sout0, sout1):
    wid = lax.axis_index("s") * NUM_CORES + lax.axis_index("c")
    row_base = wid * ROWS_PER_TILE
    pltpu.sync_copy(perm_hbm, perm_v)

    def in_slice(c):
        return x_hbm.at[pl.ds(row_base + c * CHUNK_ROWS, CHUNK_ROWS)]

    def out_slice(c):
        return y_hbm.at[pl.ds(row_base + c * CHUNK_ROWS, CHUNK_ROWS)]

    def permute_chunk(in_ref, out_ref):
        @plsc.parallel_loop(0, WIDTH // LANES, unroll=4)
        def _col_loop(jc):
            cv = perm_v[pl.ds(jc * LANES, LANES)]
            off = jc * LANES
            for r in range(CHUNK_ROWS):
                rv = jnp.full((LANES,), r, jnp.int32)
                vals = plsc.load_gather(in_ref, [rv, cv])
                out_ref[r, pl.ds(off, LANES)] = vals

    pltpu.async_copy(in_slice(0), in0, sin0)

    @pl.loop(0, NUM_CHUNKS, step=2)
    def _chunk_loop(c):
        pltpu.async_copy(in_slice(c + 1), in1, sin1)
        pltpu.make_async_copy(in_slice(c), in0, sin0).wait()

        @pl.when(c >= 2)
        def _drain_out0():
            pltpu.make_async_copy(out0, out_slice(c), sout0).wait()

        permute_chunk(in0, out0)
        pltpu.async_copy(out0, out_slice(c), sout0)

        @pl.when(c + 2 < NUM_CHUNKS)
        def _prefetch_in0():
            pltpu.async_copy(in_slice(c + 2), in0, sin0)

        pltpu.make_async_copy(in_slice(c + 1), in1, sin1).wait()

        @pl.when(c >= 2)
        def _drain_out1():
            pltpu.make_async_copy(out1, out_slice(c + 1), sout1).wait()

        permute_chunk(in1, out1)
        pltpu.async_copy(out1, out_slice(c + 1), sout1)

    pltpu.make_async_copy(out0, out_slice(NUM_CHUNKS - 2), sout0).wait()
    pltpu.make_async_copy(out1, out_slice(NUM_CHUNKS - 1), sout1).wait()


def kernel(x, perm):
    mesh = plsc.VectorSubcoreMesh(
        core_axis_name="c", subcore_axis_name="s",
        num_cores=NUM_CORES, num_subcores=NUM_SUBCORES)
    run = pl.kernel(
        _sc_body,
        out_type=jax.ShapeDtypeStruct((BATCH, WIDTH), jnp.float32),
        mesh=mesh,
        scratch_types=[
            pltpu.VMEM((WIDTH,), jnp.int32),
            pltpu.VMEM((CHUNK_ROWS, WIDTH), jnp.float32),
            pltpu.VMEM((CHUNK_ROWS, WIDTH), jnp.float32),
            pltpu.VMEM((CHUNK_ROWS, WIDTH), jnp.float32),
            pltpu.VMEM((CHUNK_ROWS, WIDTH), jnp.float32),
            pltpu.SemaphoreType.DMA,
            pltpu.SemaphoreType.DMA,
            pltpu.SemaphoreType.DMA,
            pltpu.SemaphoreType.DMA,
        ],
        compiler_params=pltpu.CompilerParams(
            needs_layout_passes=False, use_tc_tiling_on_sc=True),
    )
    y = run(x, perm.astype(jnp.int32))
    return (y, 0.0)


# speedup vs baseline: 7.2978x; 1.0051x over previous
import jax
import jax.numpy as jnp
from jax import lax
from jax.experimental import pallas as pl
from jax.experimental.pallas import tpu as pltpu
from jax.experimental.pallas import tpu_sc as plsc

BATCH = 8192
WIDTH = 2048
LANES = 16
NUM_CORES = 2
NUM_SUBCORES = 16
NUM_TILES = NUM_CORES * NUM_SUBCORES
ROWS_PER_TILE = BATCH // NUM_TILES
CHUNK_ROWS = 8
NUM_CHUNKS = ROWS_PER_TILE // CHUNK_ROWS


def _sc_body(x_hbm, perm_hbm, y_hbm, perm_v, in0, in1, out0, out1,
             sin0, sin1, sout0, sout1):
    wid = lax.axis_index("s") * NUM_CORES + lax.axis_index("c")
    row_base = wid * ROWS_PER_TILE
    pltpu.sync_copy(perm_hbm, perm_v)

    def in_slice(c):
        return x_hbm.at[pl.ds(row_base + c * CHUNK_ROWS, CHUNK_ROWS)]

    def out_slice(c):
        return y_hbm.at[pl.ds(row_base + c * CHUNK_ROWS, CHUNK_ROWS)]

    def permute_chunk(in_ref, out_ref):
        @plsc.parallel_loop(0, WIDTH // LANES, unroll=8)
        def _col_loop(jc):
            cv = perm_v[pl.ds(jc * LANES, LANES)]
            off = jc * LANES
            for r in range(CHUNK_ROWS):
                rv = jnp.full((LANES,), r, jnp.int32)
                vals = plsc.load_gather(in_ref, [rv, cv])
                out_ref[r, pl.ds(off, LANES)] = vals

    pltpu.async_copy(in_slice(0), in0, sin0)

    @pl.loop(0, NUM_CHUNKS, step=2)
    def _chunk_loop(c):
        pltpu.async_copy(in_slice(c + 1), in1, sin1)
        pltpu.make_async_copy(in_slice(c), in0, sin0).wait()

        @pl.when(c >= 2)
        def _drain_out0():
            pltpu.make_async_copy(out0, out_slice(c), sout0).wait()

        permute_chunk(in0, out0)
        pltpu.async_copy(out0, out_slice(c), sout0)

        @pl.when(c + 2 < NUM_CHUNKS)
        def _prefetch_in0():
            pltpu.async_copy(in_slice(c + 2), in0, sin0)

        pltpu.make_async_copy(in_slice(c + 1), in1, sin1).wait()

        @pl.when(c >= 2)
        def _drain_out1():
            pltpu.make_async_copy(out1, out_slice(c + 1), sout1).wait()

        permute_chunk(in1, out1)
        pltpu.async_copy(out1, out_slice(c + 1), sout1)

    pltpu.make_async_copy(out0, out_slice(NUM_CHUNKS - 2), sout0).wait()
    pltpu.make_async_copy(out1, out_slice(NUM_CHUNKS - 1), sout1).wait()


def kernel(x, perm):
    mesh = plsc.VectorSubcoreMesh(
        core_axis_name="c", subcore_axis_name="s",
        num_cores=NUM_CORES, num_subcores=NUM_SUBCORES)
    run = pl.kernel(
        _sc_body,
        out_type=jax.ShapeDtypeStruct((BATCH, WIDTH), jnp.float32),
        mesh=mesh,
        scratch_types=[
            pltpu.VMEM((WIDTH,), jnp.int32),
            pltpu.VMEM((CHUNK_ROWS, WIDTH), jnp.float32),
            pltpu.VMEM((CHUNK_ROWS, WIDTH), jnp.float32),
            pltpu.VMEM((CHUNK_ROWS, WIDTH), jnp.float32),
            pltpu.VMEM((CHUNK_ROWS, WIDTH), jnp.float32),
            pltpu.SemaphoreType.DMA,
            pltpu.SemaphoreType.DMA,
            pltpu.SemaphoreType.DMA,
            pltpu.SemaphoreType.DMA,
        ],
        compiler_params=pltpu.CompilerParams(
            needs_layout_passes=False, use_tc_tiling_on_sc=True),
    )
    y = run(x, perm.astype(jnp.int32))
    return (y, 0.0)


# speedup vs baseline: 7.4769x; 1.0245x over previous
import jax
import jax.numpy as jnp
from jax import lax
from jax.experimental import pallas as pl
from jax.experimental.pallas import tpu as pltpu
from jax.experimental.pallas import tpu_sc as plsc

BATCH = 8192
WIDTH = 2048
LANES = 16
NUM_CORES = 2
NUM_SUBCORES = 16
NUM_TILES = NUM_CORES * NUM_SUBCORES
ROWS_PER_TILE = BATCH // NUM_TILES
CHUNK_ROWS = 8
NUM_CHUNKS = ROWS_PER_TILE // CHUNK_ROWS
NBUF_IN = 4
NBUF_OUT = 2


def _sc_body(x_hbm, perm_hbm, y_hbm, perm_v, *bufs):
    ins = bufs[:NBUF_IN]
    outs = bufs[NBUF_IN:NBUF_IN + NBUF_OUT]
    sins = bufs[NBUF_IN + NBUF_OUT:NBUF_IN + NBUF_OUT + NBUF_IN]
    souts = bufs[NBUF_IN + NBUF_OUT + NBUF_IN:]

    wid = lax.axis_index("s") * NUM_CORES + lax.axis_index("c")
    row_base = wid * ROWS_PER_TILE
    pltpu.sync_copy(perm_hbm, perm_v)

    def in_slice(c):
        return x_hbm.at[pl.ds(row_base + c * CHUNK_ROWS, CHUNK_ROWS)]

    def out_slice(c):
        return y_hbm.at[pl.ds(row_base + c * CHUNK_ROWS, CHUNK_ROWS)]

    def permute_chunk(in_ref, out_ref):
        @plsc.parallel_loop(0, WIDTH // LANES, unroll=8)
        def _col_loop(jc):
            cv = perm_v[pl.ds(jc * LANES, LANES)]
            off = jc * LANES
            for r in range(CHUNK_ROWS):
                rv = jnp.full((LANES,), r, jnp.int32)
                vals = plsc.load_gather(in_ref, [rv, cv])
                out_ref[r, pl.ds(off, LANES)] = vals

    for b in range(NBUF_IN - 1):
        pltpu.async_copy(in_slice(b), ins[b], sins[b])

    @pl.loop(0, NUM_CHUNKS, step=NBUF_IN)
    def _chunk_loop(c):
        for b in range(NBUF_IN):
            cc = c + b
            pf = cc + NBUF_IN - 1
            pb = (b + NBUF_IN - 1) % NBUF_IN

            @pl.when(pf < NUM_CHUNKS)
            def _prefetch():
                pltpu.async_copy(in_slice(pf), ins[pb], sins[pb])

            pltpu.make_async_copy(in_slice(cc), ins[b], sins[b]).wait()
            ob = b % NBUF_OUT

            @pl.when(cc >= NBUF_OUT)
            def _drain_out():
                pltpu.make_async_copy(outs[ob], out_slice(cc), souts[ob]).wait()

            permute_chunk(ins[b], outs[ob])
            pltpu.async_copy(outs[ob], out_slice(cc), souts[ob])

    for b in range(NBUF_OUT):
        pltpu.make_async_copy(
            outs[b], out_slice(NUM_CHUNKS - NBUF_OUT + b), souts[b]).wait()


def kernel(x, perm):
    mesh = plsc.VectorSubcoreMesh(
        core_axis_name="c", subcore_axis_name="s",
        num_cores=NUM_CORES, num_subcores=NUM_SUBCORES)
    run = pl.kernel(
        _sc_body,
        out_type=jax.ShapeDtypeStruct((BATCH, WIDTH), jnp.float32),
        mesh=mesh,
        scratch_types=(
            [pltpu.VMEM((WIDTH,), jnp.int32)]
            + [pltpu.VMEM((CHUNK_ROWS, WIDTH), jnp.float32)] * (NBUF_IN + NBUF_OUT)
            + [pltpu.SemaphoreType.DMA] * (NBUF_IN + NBUF_OUT)
        ),
        compiler_params=pltpu.CompilerParams(
            needs_layout_passes=False, use_tc_tiling_on_sc=True),
    )
    y = run(x, perm.astype(jnp.int32))
    return (y, 0.0)
